# baseline (speedup 1.0000x reference)
import jax
import jax.numpy as jnp
from jax.experimental import pallas as pl
from jax.experimental.pallas import tpu as pltpu

_B = 16384
_A = 26
_V = 128
_ROWS = 256


def _loss_kernel(si_ref, ro_ref, loss_ref, acc_ref, accor_ref):
    i = pl.program_id(0)
    x = ro_ref[...]
    si = si_ref[...]
    x3 = x.reshape(_ROWS, _A, _V)

    sp = jnp.maximum(x, 0.0) + jnp.log1p(jnp.exp(-jnp.abs(x)))
    s_sp = jnp.sum(sp)

    iota_v = jax.lax.broadcasted_iota(jnp.int32, (_ROWS, _A, _V), 2)
    onehot = iota_v == si[:, :, None]
    s_gather = jnp.sum(jnp.where(onehot, x3, 0.0))

    m = jnp.max(x3, axis=2, keepdims=True)
    idx = jnp.min(jnp.where(x3 == m, iota_v, _V), axis=2)
    correct = idx == si
    s_accor = jnp.sum(correct.astype(jnp.float32))
    s_acc = jnp.sum((jnp.sum(correct.astype(jnp.int32), axis=1) == _A)
                    .astype(jnp.float32))

    zero = jnp.zeros((1, 1), jnp.float32)

    @pl.when(i == 0)
    def _init():
        loss_ref[...] = zero
        acc_ref[...] = zero
        accor_ref[...] = zero

    loss_ref[...] += (s_sp - s_gather).reshape(1, 1)
    acc_ref[...] += s_acc.reshape(1, 1)
    accor_ref[...] += s_accor.reshape(1, 1)


def kernel(sender_input, _message, _receiver_input, receiver_output, _labels):
    n_blocks = _B // _ROWS
    out_shape = [jax.ShapeDtypeStruct((1, 1), jnp.float32)] * 3
    loss_sum, acc_sum, accor_sum = pl.pallas_call(
        _loss_kernel,
        grid=(n_blocks,),
        in_specs=[
            pl.BlockSpec((_ROWS, _A), lambda i: (i, 0)),
            pl.BlockSpec((_ROWS, _A * _V), lambda i: (i, 0)),
        ],
        out_specs=[pl.BlockSpec((1, 1), lambda i: (0, 0))] * 3,
        out_shape=out_shape,
    )(sender_input, receiver_output)
    denom = jnp.float32(_B * _A * _V)
    loss = loss_sum[0, 0] / denom
    acc = acc_sum[0, 0] / jnp.float32(_B)
    acc_or = accor_sum[0, 0] / jnp.float32(_B * _A)
    return (loss, acc, acc_or)
